# Initial kernel scaffold; baseline (speedup 1.0000x reference)
import jax
import jax.numpy as jnp
from jax.experimental import pallas as pl


def kernel(x, W1, b1, W2, b2, mlp_W1, mlp_b1, mlp_W2, mlp_b2, proj_W, proj_b):
    raise NotImplementedError("write your pallas kernel here")


# speedup vs baseline: 21.3935x; 21.3935x over previous
import numpy as np
import jax
import jax.numpy as jnp
from jax.experimental import pallas as pl
from jax.experimental.pallas import tpu as pltpu

_N = 400
_B = 16
_HC = 32
_TRIU1 = _N * (_N - 1) // 2
_TRIU0 = _N * (_N + 1) // 2
_K = max(1, int(_TRIU1 * 0.05))
_EPS = 1e-5
_S = float(np.sqrt(1.0 + _EPS))
_IU0, _JU0 = np.triu_indices(_N, 0)

_KBLK = 1024
_IN_MLP = _TRIU0 + 2 * _HC
_NKB = -(-_IN_MLP // _KBLK)
_PADDED = _NKB * _KBLK


def _graph_kernel(x_ref, w1_ref, b1_ref, w2_ref, b2_ref, hm_ref):
    xb = x_ref[0]
    bits = jax.lax.bitcast_convert_type(xb, jnp.int32)
    key = bits ^ jnp.where(bits < 0, jnp.int32(0x7FFFFFFF), jnp.int32(0))
    row = jax.lax.broadcasted_iota(jnp.int32, (_N, _N), 0)
    col = jax.lax.broadcasted_iota(jnp.int32, (_N, _N), 1)
    upper = col > row
    neg_inf = jnp.int32(-2147483648)
    key_m = jnp.where(upper, key, neg_inf)

    def bit_step(i, t):
        cand = t + (jnp.int32(1) << (jnp.int32(31) - i))
        cnt = jnp.sum((key_m >= cand).astype(jnp.int32))
        return jnp.where(cnt >= _K, cand, t)

    t = jax.lax.fori_loop(0, 32, bit_step, neg_inf)
    cnt_gt = jnp.sum((key_m > t).astype(jnp.int32))
    need_eq = _K - cnt_gt

    eq = (key_m == t).astype(jnp.float32)
    upper_f = upper.astype(jnp.float32)
    cum_in_row = jax.lax.dot_general(
        eq, upper_f, (((1,), (0,)), ((), ())),
        preferred_element_type=jnp.float32)
    row_tot = jnp.sum(eq, axis=1)
    base = jax.lax.dot_general(
        row_tot, upper_f, (((0,), (0,)), ((), ())),
        preferred_element_type=jnp.float32)
    rank = base[:, None] + cum_in_row
    keep_tie = (eq > 0.0) & (rank < need_eq.astype(jnp.float32))
    sel = jnp.where((key_m > t) | keep_tie, 1.0, 0.0)

    asym = sel + sel.T + jnp.where(col == row, 1.0, 0.0)
    deg = jnp.sum(sel + sel.T, axis=1) + 1.0
    dinv = jax.lax.rsqrt(deg)
    anorm = dinv[:, None] * asym * dinv[None, :]

    xw1 = jax.lax.dot_general(xb, w1_ref[...], (((1,), (1,)), ((), ())),
                              preferred_element_type=jnp.float32)
    h1 = jnp.tanh(jax.lax.dot_general(anorm, xw1, (((1,), (0,)), ((), ())),
                                      preferred_element_type=jnp.float32)
                  + b1_ref[0][None, :])
    xw2 = jax.lax.dot_general(h1, w2_ref[...], (((1,), (1,)), ((), ())),
                              preferred_element_type=jnp.float32)
    h2 = jnp.tanh(jax.lax.dot_general(anorm, xw2, (((1,), (0,)), ((), ())),
                                      preferred_element_type=jnp.float32)
                  + b2_ref[0][None, :])
    hm_ref[0, 0] = jnp.concatenate(
        [jnp.mean(h1, axis=0), jnp.mean(h2, axis=0)], axis=0)


def _mlp_kernel(f_ref, w_ref, w2m_ref, b2m_ref, wp_ref, bp_ref, b1m_ref,
                out_ref, acc_ref):
    k = pl.program_id(0)

    @pl.when(k == 0)
    def _():
        acc_ref[...] = jnp.zeros_like(acc_ref)

    col = jax.lax.broadcasted_iota(jnp.int32, (64, _KBLK), 1)
    limit = jnp.int32(_IN_MLP) - k * _KBLK
    wblk = jnp.where(col < limit, w_ref[...], 0.0)
    acc_ref[...] += jax.lax.dot_general(
        f_ref[...], wblk, (((1,), (1,)), ((), ())),
        preferred_element_type=jnp.float32)

    @pl.when(k == _NKB - 1)
    def _():
        z1 = jax.nn.relu(acc_ref[...] * (1.0 / (1.0 + _EPS))
                         + b1m_ref[0][None, :] * (1.0 / _S))
        z2 = jax.lax.dot_general(z1, w2m_ref[...], (((1,), (1,)), ((), ())),
                                 preferred_element_type=jnp.float32)
        z2 = jax.nn.relu((z2 + b2m_ref[0][None, :]) * (1.0 / _S))
        z3 = jax.lax.dot_general(z2, wp_ref[...], (((1,), (1,)), ((), ())),
                                 preferred_element_type=jnp.float32)
        out_ref[...] = jax.nn.relu(z3 + bp_ref[0][None, :])


def kernel(x, W1, b1, W2, b2, mlp_W1, mlp_b1, mlp_W2, mlp_b2, proj_W, proj_b):
    hm = pl.pallas_call(
        _graph_kernel,
        grid=(_B,),
        in_specs=[
            pl.BlockSpec((1, _N, _N), lambda b: (b, 0, 0)),
            pl.BlockSpec((_HC, _N), lambda b: (0, 0)),
            pl.BlockSpec((1, _HC), lambda b: (0, 0)),
            pl.BlockSpec((_HC, _HC), lambda b: (0, 0)),
            pl.BlockSpec((1, _HC), lambda b: (0, 0)),
        ],
        out_specs=pl.BlockSpec((1, 1, 2 * _HC), lambda b: (b, 0, 0)),
        out_shape=jax.ShapeDtypeStruct((_B, 1, 2 * _HC), jnp.float32),
    )(x, W1, b1.reshape(1, _HC), W2, b2.reshape(1, _HC))

    triu_feats = x[:, _IU0, _JU0]
    feats = jnp.concatenate(
        [triu_feats, hm.reshape(_B, 2 * _HC),
         jnp.zeros((_B, _PADDED - _IN_MLP), jnp.float32)], axis=1)

    out = pl.pallas_call(
        _mlp_kernel,
        grid=(_NKB,),
        in_specs=[
            pl.BlockSpec((_B, _KBLK), lambda k: (0, k)),
            pl.BlockSpec((64, _KBLK), lambda k: (0, k)),
            pl.BlockSpec((32, 64), lambda k: (0, 0)),
            pl.BlockSpec((1, 32), lambda k: (0, 0)),
            pl.BlockSpec((256, 32), lambda k: (0, 0)),
            pl.BlockSpec((1, 256), lambda k: (0, 0)),
            pl.BlockSpec((1, 64), lambda k: (0, 0)),
        ],
        out_specs=pl.BlockSpec((_B, 256), lambda k: (0, 0)),
        out_shape=jax.ShapeDtypeStruct((_B, 256), jnp.float32),
        scratch_shapes=[pltpu.VMEM((_B, 64), jnp.float32)],
    )(feats, mlp_W1, mlp_W2, mlp_b2.reshape(1, 32), proj_W,
      proj_b.reshape(1, 256), mlp_b1.reshape(1, 64))
    return out


# speedup vs baseline: 22.4723x; 1.0504x over previous
import numpy as np
import jax
import jax.numpy as jnp
from jax.experimental import pallas as pl
from jax.experimental.pallas import tpu as pltpu

_N = 400
_B = 16
_HC = 32
_TRIU1 = _N * (_N - 1) // 2
_TRIU0 = _N * (_N + 1) // 2
_K = max(1, int(_TRIU1 * 0.05))
_EPS = 1e-5
_S = float(np.sqrt(1.0 + _EPS))
_IU0, _JU0 = np.triu_indices(_N, 0)

_KBLK = 1024
_NKB = -(-_TRIU0 // _KBLK)


def _graph_kernel(x_ref, w1_ref, b1_ref, w2_ref, b2_ref, hm_ref):
    xb = x_ref[0]
    bits = jax.lax.bitcast_convert_type(xb, jnp.int32)
    key = bits ^ jnp.where(bits < 0, jnp.int32(0x7FFFFFFF), jnp.int32(0))
    row = jax.lax.broadcasted_iota(jnp.int32, (_N, _N), 0)
    col = jax.lax.broadcasted_iota(jnp.int32, (_N, _N), 1)
    upper = col > row
    neg_inf = jnp.int32(-2147483648)
    key_m = jnp.where(upper, key, neg_inf)

    def bit_step(i, t):
        cand = t + (jnp.int32(1) << (jnp.int32(31) - i))
        cnt = jnp.sum((key_m >= cand).astype(jnp.int32))
        return jnp.where(cnt >= _K, cand, t)

    t = jax.lax.fori_loop(0, 32, bit_step, neg_inf)
    cnt_gt = jnp.sum((key_m > t).astype(jnp.int32))
    need_eq = _K - cnt_gt

    eq = (key_m == t).astype(jnp.float32)
    upper_f = upper.astype(jnp.float32)
    cum_in_row = jax.lax.dot_general(
        eq, upper_f, (((1,), (0,)), ((), ())),
        preferred_element_type=jnp.float32)
    row_tot = jnp.sum(eq, axis=1)
    base = jax.lax.dot_general(
        row_tot, upper_f, (((0,), (0,)), ((), ())),
        preferred_element_type=jnp.float32)
    rank = base[:, None] + cum_in_row
    keep_tie = (eq > 0.0) & (rank < need_eq.astype(jnp.float32))
    sel = jnp.where((key_m > t) | keep_tie, 1.0, 0.0)

    asym = sel + sel.T + jnp.where(col == row, 1.0, 0.0)
    deg = jnp.sum(sel + sel.T, axis=1) + 1.0
    dinv = jax.lax.rsqrt(deg)
    anorm = dinv[:, None] * asym * dinv[None, :]

    xw1 = jax.lax.dot_general(xb, w1_ref[...], (((1,), (1,)), ((), ())),
                              preferred_element_type=jnp.float32)
    h1 = jnp.tanh(jax.lax.dot_general(anorm, xw1, (((1,), (0,)), ((), ())),
                                      preferred_element_type=jnp.float32)
                  + b1_ref[0][None, :])
    xw2 = jax.lax.dot_general(h1, w2_ref[...], (((1,), (1,)), ((), ())),
                              preferred_element_type=jnp.float32)
    h2 = jnp.tanh(jax.lax.dot_general(anorm, xw2, (((1,), (0,)), ((), ())),
                                      preferred_element_type=jnp.float32)
                  + b2_ref[0][None, :])
    hm_ref[0, 0] = jnp.concatenate(
        [jnp.mean(h1, axis=0), jnp.mean(h2, axis=0)], axis=0)


def _mlp_kernel(f_ref, w_ref, hm_ref, wh_ref, w2m_ref, b2m_ref, wp_ref,
                bp_ref, b1m_ref, out_ref, acc_ref):
    k = pl.program_id(0)

    @pl.when(k == 0)
    def _():
        acc_ref[...] = jnp.zeros_like(acc_ref)

    limit = jnp.int32(_TRIU0) - k * _KBLK
    col = jax.lax.broadcasted_iota(jnp.int32, (64, _KBLK), 1)
    wblk = jnp.where(col < limit, w_ref[...], 0.0)
    colf = jax.lax.broadcasted_iota(jnp.int32, (_B, _KBLK), 1)
    fblk = jnp.where(colf < limit, f_ref[...], 0.0)
    acc_ref[...] += jax.lax.dot_general(
        fblk, wblk, (((1,), (1,)), ((), ())),
        preferred_element_type=jnp.float32)

    @pl.when(k == _NKB - 1)
    def _():
        acc = acc_ref[...] + jax.lax.dot_general(
            hm_ref[...], wh_ref[...], (((1,), (1,)), ((), ())),
            preferred_element_type=jnp.float32)
        z1 = jax.nn.relu(acc * (1.0 / (1.0 + _EPS))
                         + b1m_ref[0][None, :] * (1.0 / _S))
        z2 = jax.lax.dot_general(z1, w2m_ref[...], (((1,), (1,)), ((), ())),
                                 preferred_element_type=jnp.float32)
        z2 = jax.nn.relu((z2 + b2m_ref[0][None, :]) * (1.0 / _S))
        z3 = jax.lax.dot_general(z2, wp_ref[...], (((1,), (1,)), ((), ())),
                                 preferred_element_type=jnp.float32)
        out_ref[...] = jax.nn.relu(z3 + bp_ref[0][None, :])


def kernel(x, W1, b1, W2, b2, mlp_W1, mlp_b1, mlp_W2, mlp_b2, proj_W, proj_b):
    hm = pl.pallas_call(
        _graph_kernel,
        grid=(_B,),
        in_specs=[
            pl.BlockSpec((1, _N, _N), lambda b: (b, 0, 0)),
            pl.BlockSpec((_HC, _N), lambda b: (0, 0)),
            pl.BlockSpec((1, _HC), lambda b: (0, 0)),
            pl.BlockSpec((_HC, _HC), lambda b: (0, 0)),
            pl.BlockSpec((1, _HC), lambda b: (0, 0)),
        ],
        out_specs=pl.BlockSpec((1, 1, 2 * _HC), lambda b: (b, 0, 0)),
        out_shape=jax.ShapeDtypeStruct((_B, 1, 2 * _HC), jnp.float32),
    )(x, W1, b1.reshape(1, _HC), W2, b2.reshape(1, _HC))

    triu_feats = x[:, _IU0, _JU0]

    out = pl.pallas_call(
        _mlp_kernel,
        grid=(_NKB,),
        in_specs=[
            pl.BlockSpec((_B, _KBLK), lambda k: (0, k)),
            pl.BlockSpec((64, _KBLK), lambda k: (0, k)),
            pl.BlockSpec((_B, 2 * _HC), lambda k: (0, 0)),
            pl.BlockSpec((64, 2 * _HC), lambda k: (0, 0)),
            pl.BlockSpec((32, 64), lambda k: (0, 0)),
            pl.BlockSpec((1, 32), lambda k: (0, 0)),
            pl.BlockSpec((256, 32), lambda k: (0, 0)),
            pl.BlockSpec((1, 256), lambda k: (0, 0)),
            pl.BlockSpec((1, 64), lambda k: (0, 0)),
        ],
        out_specs=pl.BlockSpec((_B, 256), lambda k: (0, 0)),
        out_shape=jax.ShapeDtypeStruct((_B, 256), jnp.float32),
        scratch_shapes=[pltpu.VMEM((_B, 64), jnp.float32)],
    )(triu_feats, mlp_W1, hm.reshape(_B, 2 * _HC), mlp_W1[:, _TRIU0:],
      mlp_W2, mlp_b2.reshape(1, 32), proj_W, proj_b.reshape(1, 256),
      mlp_b1.reshape(1, 64))
    return out


# speedup vs baseline: 32.7061x; 1.4554x over previous
import numpy as np
import jax
import jax.numpy as jnp
from jax.experimental import pallas as pl
from jax.experimental.pallas import tpu as pltpu

_N = 400
_B = 16
_HC = 32
_TRIU1 = _N * (_N - 1) // 2
_TRIU0 = _N * (_N + 1) // 2
_K = max(1, int(_TRIU1 * 0.05))
_EPS = 1e-5
_S = float(np.sqrt(1.0 + _EPS))
_IU0, _JU0 = np.triu_indices(_N, 0)

_KBLK = 2048
_NKB = -(-_TRIU0 // _KBLK)


def _thresh_kernel(x_ref, tn_ref):
    xb = x_ref[...]
    bits = jax.lax.bitcast_convert_type(xb, jnp.int32)
    key = bits ^ jnp.where(bits < 0, jnp.int32(0x7FFFFFFF), jnp.int32(0))
    row = jax.lax.broadcasted_iota(jnp.int32, (_B, _N, _N), 1)
    col = jax.lax.broadcasted_iota(jnp.int32, (_B, _N, _N), 2)
    neg_inf = jnp.int32(-2147483648)
    key_m = jnp.where(col > row, key, neg_inf)

    def bit_step(i, t):
        cand = t + (jnp.int32(1) << (jnp.int32(31) - i))
        cnt = jnp.sum((key_m >= cand[:, None, None]).astype(jnp.int32),
                      axis=(1, 2))
        return jnp.where(cnt >= _K, cand, t)

    t = jax.lax.fori_loop(
        0, 32, bit_step, jnp.full((_B,), neg_inf, jnp.int32))
    cnt_gt = jnp.sum((key_m > t[:, None, None]).astype(jnp.int32),
                     axis=(1, 2))
    need_eq = _K - cnt_gt
    ci = jax.lax.broadcasted_iota(jnp.int32, (_B, 8), 1)
    tn_ref[...] = jnp.where(ci == 0, t[:, None],
                            jnp.where(ci == 1, need_eq[:, None], 0))


def _graph_kernel(x_ref, tn_ref, w1_ref, b1_ref, w2_ref, b2_ref, hm_ref):
    xb = x_ref[0]
    bits = jax.lax.bitcast_convert_type(xb, jnp.int32)
    key = bits ^ jnp.where(bits < 0, jnp.int32(0x7FFFFFFF), jnp.int32(0))
    row = jax.lax.broadcasted_iota(jnp.int32, (_N, _N), 0)
    col = jax.lax.broadcasted_iota(jnp.int32, (_N, _N), 1)
    upper = col > row
    neg_inf = jnp.int32(-2147483648)
    key_m = jnp.where(upper, key, neg_inf)

    t = tn_ref[0, 0, 0]
    need_eq = tn_ref[0, 0, 1]

    eq = (key_m == t).astype(jnp.float32)
    upper_f = upper.astype(jnp.float32)
    cum_in_row = jax.lax.dot_general(
        eq, upper_f, (((1,), (0,)), ((), ())),
        preferred_element_type=jnp.float32)
    row_tot = jnp.sum(eq, axis=1)
    base = jax.lax.dot_general(
        row_tot, upper_f, (((0,), (0,)), ((), ())),
        preferred_element_type=jnp.float32)
    rank = base[:, None] + cum_in_row
    keep_tie = (eq > 0.0) & (rank < need_eq.astype(jnp.float32))
    sel = jnp.where((key_m > t) | keep_tie, 1.0, 0.0)

    asym = sel + sel.T + jnp.where(col == row, 1.0, 0.0)
    deg = jnp.sum(sel + sel.T, axis=1) + 1.0
    dinv = jax.lax.rsqrt(deg)
    anorm = dinv[:, None] * asym * dinv[None, :]

    xw1 = jax.lax.dot_general(xb, w1_ref[...], (((1,), (1,)), ((), ())),
                              preferred_element_type=jnp.float32)
    h1 = jnp.tanh(jax.lax.dot_general(anorm, xw1, (((1,), (0,)), ((), ())),
                                      preferred_element_type=jnp.float32)
                  + b1_ref[0][None, :])
    xw2 = jax.lax.dot_general(h1, w2_ref[...], (((1,), (1,)), ((), ())),
                              preferred_element_type=jnp.float32)
    h2 = jnp.tanh(jax.lax.dot_general(anorm, xw2, (((1,), (0,)), ((), ())),
                                      preferred_element_type=jnp.float32)
                  + b2_ref[0][None, :])
    hm_ref[0, 0] = jnp.concatenate(
        [jnp.mean(h1, axis=0), jnp.mean(h2, axis=0)], axis=0)


def _mlp_kernel(f_ref, w_ref, hm_ref, wh_ref, w2m_ref, b2m_ref, wp_ref,
                bp_ref, b1m_ref, out_ref, acc_ref):
    k = pl.program_id(0)

    @pl.when(k == 0)
    def _():
        acc_ref[...] = jnp.zeros_like(acc_ref)

    limit = jnp.int32(_TRIU0) - k * _KBLK
    col = jax.lax.broadcasted_iota(jnp.int32, (64, _KBLK), 1)
    wblk = jnp.where(col < limit, w_ref[...], 0.0)
    colf = jax.lax.broadcasted_iota(jnp.int32, (_B, _KBLK), 1)
    fblk = jnp.where(colf < limit, f_ref[...], 0.0)
    acc_ref[...] += jax.lax.dot_general(
        fblk, wblk, (((1,), (1,)), ((), ())),
        preferred_element_type=jnp.float32)

    @pl.when(k == _NKB - 1)
    def _():
        acc = acc_ref[...] + jax.lax.dot_general(
            hm_ref[...], wh_ref[...], (((1,), (1,)), ((), ())),
            preferred_element_type=jnp.float32)
        z1 = jax.nn.relu(acc * (1.0 / (1.0 + _EPS))
                         + b1m_ref[0][None, :] * (1.0 / _S))
        z2 = jax.lax.dot_general(z1, w2m_ref[...], (((1,), (1,)), ((), ())),
                                 preferred_element_type=jnp.float32)
        z2 = jax.nn.relu((z2 + b2m_ref[0][None, :]) * (1.0 / _S))
        z3 = jax.lax.dot_general(z2, wp_ref[...], (((1,), (1,)), ((), ())),
                                 preferred_element_type=jnp.float32)
        out_ref[...] = jax.nn.relu(z3 + bp_ref[0][None, :])


def kernel(x, W1, b1, W2, b2, mlp_W1, mlp_b1, mlp_W2, mlp_b2, proj_W, proj_b):
    tn = pl.pallas_call(
        _thresh_kernel,
        out_shape=jax.ShapeDtypeStruct((_B, 8), jnp.int32),
    )(x)

    hm = pl.pallas_call(
        _graph_kernel,
        grid=(_B,),
        in_specs=[
            pl.BlockSpec((1, _N, _N), lambda b: (b, 0, 0)),
            pl.BlockSpec((1, 1, 8), lambda b: (b, 0, 0)),
            pl.BlockSpec((_HC, _N), lambda b: (0, 0)),
            pl.BlockSpec((1, _HC), lambda b: (0, 0)),
            pl.BlockSpec((_HC, _HC), lambda b: (0, 0)),
            pl.BlockSpec((1, _HC), lambda b: (0, 0)),
        ],
        out_specs=pl.BlockSpec((1, 1, 2 * _HC), lambda b: (b, 0, 0)),
        out_shape=jax.ShapeDtypeStruct((_B, 1, 2 * _HC), jnp.float32),
    )(x, tn.reshape(_B, 1, 8), W1, b1.reshape(1, _HC), W2,
      b2.reshape(1, _HC))

    triu_feats = x[:, _IU0, _JU0]

    out = pl.pallas_call(
        _mlp_kernel,
        grid=(_NKB,),
        in_specs=[
            pl.BlockSpec((_B, _KBLK), lambda k: (0, k)),
            pl.BlockSpec((64, _KBLK), lambda k: (0, k)),
            pl.BlockSpec((_B, 2 * _HC), lambda k: (0, 0)),
            pl.BlockSpec((64, 2 * _HC), lambda k: (0, 0)),
            pl.BlockSpec((32, 64), lambda k: (0, 0)),
            pl.BlockSpec((1, 32), lambda k: (0, 0)),
            pl.BlockSpec((256, 32), lambda k: (0, 0)),
            pl.BlockSpec((1, 256), lambda k: (0, 0)),
            pl.BlockSpec((1, 64), lambda k: (0, 0)),
        ],
        out_specs=pl.BlockSpec((_B, 256), lambda k: (0, 0)),
        out_shape=jax.ShapeDtypeStruct((_B, 256), jnp.float32),
        scratch_shapes=[pltpu.VMEM((_B, 64), jnp.float32)],
    )(triu_feats, mlp_W1, hm.reshape(_B, 2 * _HC), mlp_W1[:, _TRIU0:],
      mlp_W2, mlp_b2.reshape(1, 32), proj_W, proj_b.reshape(1, 256),
      mlp_b1.reshape(1, 64))
    return out


# speedup vs baseline: 34.6761x; 1.0602x over previous
import numpy as np
import jax
import jax.numpy as jnp
from jax.experimental import pallas as pl
from jax.experimental.pallas import tpu as pltpu

_N = 400
_B = 16
_HC = 32
_TRIU1 = _N * (_N - 1) // 2
_TRIU0 = _N * (_N + 1) // 2
_K = max(1, int(_TRIU1 * 0.05))
_EPS = 1e-5
_S = float(np.sqrt(1.0 + _EPS))
_IU0, _JU0 = np.triu_indices(_N, 0)

_KBLK = 4096
_NKB = -(-_TRIU0 // _KBLK)


def _thresh_kernel(x_ref, tn_ref):
    xb = x_ref[...]
    bits = jax.lax.bitcast_convert_type(xb, jnp.int32)
    key = bits ^ jnp.where(bits < 0, jnp.int32(0x7FFFFFFF), jnp.int32(0))
    row = jax.lax.broadcasted_iota(jnp.int32, (_B, _N, _N), 1)
    col = jax.lax.broadcasted_iota(jnp.int32, (_B, _N, _N), 2)
    neg_inf = jnp.int32(-2147483648)
    key_m = jnp.where(col > row, key, neg_inf)

    def bit_step(i, t):
        cand = t + (jnp.int32(1) << (jnp.int32(31) - i))
        cnt = jnp.sum((key_m >= cand[:, None, None]).astype(jnp.int32),
                      axis=(1, 2))
        return jnp.where(cnt >= _K, cand, t)

    t = jax.lax.fori_loop(
        0, 32, bit_step, jnp.full((_B,), neg_inf, jnp.int32))
    cnt_gt = jnp.sum((key_m > t[:, None, None]).astype(jnp.int32),
                     axis=(1, 2))
    need_eq = _K - cnt_gt
    ci = jax.lax.broadcasted_iota(jnp.int32, (_B, 8), 1)
    tn_ref[...] = jnp.where(ci == 0, t[:, None],
                            jnp.where(ci == 1, need_eq[:, None], 0))


def _graph_kernel(x_ref, tn_ref, w1_ref, b1_ref, w2_ref, b2_ref, hm_ref):
    xb = x_ref[0]
    bits = jax.lax.bitcast_convert_type(xb, jnp.int32)
    key = bits ^ jnp.where(bits < 0, jnp.int32(0x7FFFFFFF), jnp.int32(0))
    row = jax.lax.broadcasted_iota(jnp.int32, (_N, _N), 0)
    col = jax.lax.broadcasted_iota(jnp.int32, (_N, _N), 1)
    upper = col > row
    neg_inf = jnp.int32(-2147483648)
    key_m = jnp.where(upper, key, neg_inf)

    t = tn_ref[0, 0, 0]
    need_eq = tn_ref[0, 0, 1]

    eq = (key_m == t).astype(jnp.float32)
    upper_f = upper.astype(jnp.float32)
    cum_in_row = jax.lax.dot_general(
        eq, upper_f, (((1,), (0,)), ((), ())),
        preferred_element_type=jnp.float32)
    row_tot = jnp.sum(eq, axis=1)
    base = jax.lax.dot_general(
        row_tot, upper_f, (((0,), (0,)), ((), ())),
        preferred_element_type=jnp.float32)
    rank = base[:, None] + cum_in_row
    keep_tie = (eq > 0.0) & (rank < need_eq.astype(jnp.float32))
    sel = jnp.where((key_m > t) | keep_tie, 1.0, 0.0)

    asym = sel + sel.T + jnp.where(col == row, 1.0, 0.0)
    deg = jnp.sum(sel + sel.T, axis=1) + 1.0
    dinv = jax.lax.rsqrt(deg)
    anorm = dinv[:, None] * asym * dinv[None, :]

    xw1 = jax.lax.dot_general(xb, w1_ref[...], (((1,), (1,)), ((), ())),
                              preferred_element_type=jnp.float32)
    h1 = jnp.tanh(jax.lax.dot_general(anorm, xw1, (((1,), (0,)), ((), ())),
                                      preferred_element_type=jnp.float32)
                  + b1_ref[0][None, :])
    xw2 = jax.lax.dot_general(h1, w2_ref[...], (((1,), (1,)), ((), ())),
                              preferred_element_type=jnp.float32)
    h2 = jnp.tanh(jax.lax.dot_general(anorm, xw2, (((1,), (0,)), ((), ())),
                                      preferred_element_type=jnp.float32)
                  + b2_ref[0][None, :])
    hm_ref[0, 0] = jnp.concatenate(
        [jnp.mean(h1, axis=0), jnp.mean(h2, axis=0)], axis=0)


def _mlp_kernel(f_ref, w_ref, hm_ref, wh_ref, w2m_ref, b2m_ref, wp_ref,
                bp_ref, b1m_ref, out_ref, acc_ref):
    k = pl.program_id(0)

    @pl.when(k == 0)
    def _():
        acc_ref[...] = jnp.zeros_like(acc_ref)

    limit = jnp.int32(_TRIU0) - k * _KBLK
    col = jax.lax.broadcasted_iota(jnp.int32, (64, _KBLK), 1)
    wblk = jnp.where(col < limit, w_ref[...], 0.0)
    colf = jax.lax.broadcasted_iota(jnp.int32, (_B, _KBLK), 1)
    fblk = jnp.where(colf < limit, f_ref[...], 0.0)
    acc_ref[...] += jax.lax.dot_general(
        fblk, wblk, (((1,), (1,)), ((), ())),
        preferred_element_type=jnp.float32)

    @pl.when(k == _NKB - 1)
    def _():
        acc = acc_ref[...] + jax.lax.dot_general(
            hm_ref[...], wh_ref[...], (((1,), (1,)), ((), ())),
            preferred_element_type=jnp.float32)
        z1 = jax.nn.relu(acc * (1.0 / (1.0 + _EPS))
                         + b1m_ref[0][None, :] * (1.0 / _S))
        z2 = jax.lax.dot_general(z1, w2m_ref[...], (((1,), (1,)), ((), ())),
                                 preferred_element_type=jnp.float32)
        z2 = jax.nn.relu((z2 + b2m_ref[0][None, :]) * (1.0 / _S))
        z3 = jax.lax.dot_general(z2, wp_ref[...], (((1,), (1,)), ((), ())),
                                 preferred_element_type=jnp.float32)
        out_ref[...] = jax.nn.relu(z3 + bp_ref[0][None, :])


def kernel(x, W1, b1, W2, b2, mlp_W1, mlp_b1, mlp_W2, mlp_b2, proj_W, proj_b):
    tn = pl.pallas_call(
        _thresh_kernel,
        out_shape=jax.ShapeDtypeStruct((_B, 8), jnp.int32),
    )(x)

    hm = pl.pallas_call(
        _graph_kernel,
        grid=(_B,),
        in_specs=[
            pl.BlockSpec((1, _N, _N), lambda b: (b, 0, 0)),
            pl.BlockSpec((1, 1, 8), lambda b: (b, 0, 0)),
            pl.BlockSpec((_HC, _N), lambda b: (0, 0)),
            pl.BlockSpec((1, _HC), lambda b: (0, 0)),
            pl.BlockSpec((_HC, _HC), lambda b: (0, 0)),
            pl.BlockSpec((1, _HC), lambda b: (0, 0)),
        ],
        out_specs=pl.BlockSpec((1, 1, 2 * _HC), lambda b: (b, 0, 0)),
        out_shape=jax.ShapeDtypeStruct((_B, 1, 2 * _HC), jnp.float32),
    )(x, tn.reshape(_B, 1, 8), W1, b1.reshape(1, _HC), W2,
      b2.reshape(1, _HC))

    triu_feats = x[:, _IU0, _JU0]

    out = pl.pallas_call(
        _mlp_kernel,
        grid=(_NKB,),
        in_specs=[
            pl.BlockSpec((_B, _KBLK), lambda k: (0, k)),
            pl.BlockSpec((64, _KBLK), lambda k: (0, k)),
            pl.BlockSpec((_B, 2 * _HC), lambda k: (0, 0)),
            pl.BlockSpec((64, 2 * _HC), lambda k: (0, 0)),
            pl.BlockSpec((32, 64), lambda k: (0, 0)),
            pl.BlockSpec((1, 32), lambda k: (0, 0)),
            pl.BlockSpec((256, 32), lambda k: (0, 0)),
            pl.BlockSpec((1, 256), lambda k: (0, 0)),
            pl.BlockSpec((1, 64), lambda k: (0, 0)),
        ],
        out_specs=pl.BlockSpec((_B, 256), lambda k: (0, 0)),
        out_shape=jax.ShapeDtypeStruct((_B, 256), jnp.float32),
        scratch_shapes=[pltpu.VMEM((_B, 64), jnp.float32)],
    )(triu_feats, mlp_W1, hm.reshape(_B, 2 * _HC), mlp_W1[:, _TRIU0:],
      mlp_W2, mlp_b2.reshape(1, 32), proj_W, proj_b.reshape(1, 256),
      mlp_b1.reshape(1, 64))
    return out


# speedup vs baseline: 35.9240x; 1.0360x over previous
import numpy as np
import jax
import jax.numpy as jnp
from jax.experimental import pallas as pl
from jax.experimental.pallas import tpu as pltpu

_N = 400
_B = 16
_HC = 32
_TRIU1 = _N * (_N - 1) // 2
_TRIU0 = _N * (_N + 1) // 2
_K = max(1, int(_TRIU1 * 0.05))
_EPS = 1e-5
_S = float(np.sqrt(1.0 + _EPS))
_IU0, _JU0 = np.triu_indices(_N, 0)

_KBLK = 8192
_NKB = -(-_TRIU0 // _KBLK)


def _thresh_kernel(x_ref, tn_ref):
    xb = x_ref[...]
    bits = jax.lax.bitcast_convert_type(xb, jnp.int32)
    key = bits ^ jnp.where(bits < 0, jnp.int32(0x7FFFFFFF), jnp.int32(0))
    row = jax.lax.broadcasted_iota(jnp.int32, (_B, _N, _N), 1)
    col = jax.lax.broadcasted_iota(jnp.int32, (_B, _N, _N), 2)
    neg_inf = jnp.int32(-2147483648)
    key_m = jnp.where(col > row, key, neg_inf)

    def bit_step(i, t):
        cand = t + (jnp.int32(1) << (jnp.int32(31) - i))
        cnt = jnp.sum((key_m >= cand[:, None, None]).astype(jnp.int32),
                      axis=(1, 2))
        return jnp.where(cnt >= _K, cand, t)

    t = jax.lax.fori_loop(
        0, 32, bit_step, jnp.full((_B,), neg_inf, jnp.int32))
    cnt_gt = jnp.sum((key_m > t[:, None, None]).astype(jnp.int32),
                     axis=(1, 2))
    need_eq = _K - cnt_gt
    ci = jax.lax.broadcasted_iota(jnp.int32, (_B, 8), 1)
    tn_ref[...] = jnp.where(ci == 0, t[:, None],
                            jnp.where(ci == 1, need_eq[:, None], 0))


def _graph_kernel(x_ref, tn_ref, w1_ref, b1_ref, w2_ref, b2_ref, hm_ref):
    xb = x_ref[0]
    bits = jax.lax.bitcast_convert_type(xb, jnp.int32)
    key = bits ^ jnp.where(bits < 0, jnp.int32(0x7FFFFFFF), jnp.int32(0))
    row = jax.lax.broadcasted_iota(jnp.int32, (_N, _N), 0)
    col = jax.lax.broadcasted_iota(jnp.int32, (_N, _N), 1)
    upper = col > row
    neg_inf = jnp.int32(-2147483648)
    key_m = jnp.where(upper, key, neg_inf)

    t = tn_ref[0, 0, 0]
    need_eq = tn_ref[0, 0, 1]

    eq = (key_m == t).astype(jnp.float32)
    upper_f = upper.astype(jnp.float32)
    cum_in_row = jax.lax.dot_general(
        eq, upper_f, (((1,), (0,)), ((), ())),
        preferred_element_type=jnp.float32)
    row_tot = jnp.sum(eq, axis=1)
    base = jax.lax.dot_general(
        row_tot, upper_f, (((0,), (0,)), ((), ())),
        preferred_element_type=jnp.float32)
    rank = base[:, None] + cum_in_row
    keep_tie = (eq > 0.0) & (rank < need_eq.astype(jnp.float32))
    sel = jnp.where((key_m > t) | keep_tie, 1.0, 0.0)

    asym = sel + sel.T + jnp.where(col == row, 1.0, 0.0)
    deg = jnp.sum(sel + sel.T, axis=1) + 1.0
    dinv = jax.lax.rsqrt(deg)
    anorm = dinv[:, None] * asym * dinv[None, :]

    xw1 = jax.lax.dot_general(xb, w1_ref[...], (((1,), (1,)), ((), ())),
                              preferred_element_type=jnp.float32)
    h1 = jnp.tanh(jax.lax.dot_general(anorm, xw1, (((1,), (0,)), ((), ())),
                                      preferred_element_type=jnp.float32)
                  + b1_ref[0][None, :])
    xw2 = jax.lax.dot_general(h1, w2_ref[...], (((1,), (1,)), ((), ())),
                              preferred_element_type=jnp.float32)
    h2 = jnp.tanh(jax.lax.dot_general(anorm, xw2, (((1,), (0,)), ((), ())),
                                      preferred_element_type=jnp.float32)
                  + b2_ref[0][None, :])
    hm_ref[0, 0] = jnp.concatenate(
        [jnp.mean(h1, axis=0), jnp.mean(h2, axis=0)], axis=0)


def _mlp_kernel(f_ref, w_ref, hm_ref, wh_ref, w2m_ref, b2m_ref, wp_ref,
                bp_ref, b1m_ref, out_ref, acc_ref):
    k = pl.program_id(0)

    @pl.when(k == 0)
    def _():
        acc_ref[...] = jnp.zeros_like(acc_ref)

    limit = jnp.int32(_TRIU0) - k * _KBLK
    col = jax.lax.broadcasted_iota(jnp.int32, (64, _KBLK), 1)
    wblk = jnp.where(col < limit, w_ref[...], 0.0)
    colf = jax.lax.broadcasted_iota(jnp.int32, (_B, _KBLK), 1)
    fblk = jnp.where(colf < limit, f_ref[...], 0.0)
    acc_ref[...] += jax.lax.dot_general(
        fblk, wblk, (((1,), (1,)), ((), ())),
        preferred_element_type=jnp.float32)

    @pl.when(k == _NKB - 1)
    def _():
        acc = acc_ref[...] + jax.lax.dot_general(
            hm_ref[...], wh_ref[...], (((1,), (1,)), ((), ())),
            preferred_element_type=jnp.float32)
        z1 = jax.nn.relu(acc * (1.0 / (1.0 + _EPS))
                         + b1m_ref[0][None, :] * (1.0 / _S))
        z2 = jax.lax.dot_general(z1, w2m_ref[...], (((1,), (1,)), ((), ())),
                                 preferred_element_type=jnp.float32)
        z2 = jax.nn.relu((z2 + b2m_ref[0][None, :]) * (1.0 / _S))
        z3 = jax.lax.dot_general(z2, wp_ref[...], (((1,), (1,)), ((), ())),
                                 preferred_element_type=jnp.float32)
        out_ref[...] = jax.nn.relu(z3 + bp_ref[0][None, :])


def kernel(x, W1, b1, W2, b2, mlp_W1, mlp_b1, mlp_W2, mlp_b2, proj_W, proj_b):
    tn = pl.pallas_call(
        _thresh_kernel,
        out_shape=jax.ShapeDtypeStruct((_B, 8), jnp.int32),
    )(x)

    hm = pl.pallas_call(
        _graph_kernel,
        grid=(_B,),
        in_specs=[
            pl.BlockSpec((1, _N, _N), lambda b: (b, 0, 0)),
            pl.BlockSpec((1, 1, 8), lambda b: (b, 0, 0)),
            pl.BlockSpec((_HC, _N), lambda b: (0, 0)),
            pl.BlockSpec((1, _HC), lambda b: (0, 0)),
            pl.BlockSpec((_HC, _HC), lambda b: (0, 0)),
            pl.BlockSpec((1, _HC), lambda b: (0, 0)),
        ],
        out_specs=pl.BlockSpec((1, 1, 2 * _HC), lambda b: (b, 0, 0)),
        out_shape=jax.ShapeDtypeStruct((_B, 1, 2 * _HC), jnp.float32),
    )(x, tn.reshape(_B, 1, 8), W1, b1.reshape(1, _HC), W2,
      b2.reshape(1, _HC))

    triu_feats = x[:, _IU0, _JU0]

    out = pl.pallas_call(
        _mlp_kernel,
        grid=(_NKB,),
        in_specs=[
            pl.BlockSpec((_B, _KBLK), lambda k: (0, k)),
            pl.BlockSpec((64, _KBLK), lambda k: (0, k)),
            pl.BlockSpec((_B, 2 * _HC), lambda k: (0, 0)),
            pl.BlockSpec((64, 2 * _HC), lambda k: (0, 0)),
            pl.BlockSpec((32, 64), lambda k: (0, 0)),
            pl.BlockSpec((1, 32), lambda k: (0, 0)),
            pl.BlockSpec((256, 32), lambda k: (0, 0)),
            pl.BlockSpec((1, 256), lambda k: (0, 0)),
            pl.BlockSpec((1, 64), lambda k: (0, 0)),
        ],
        out_specs=pl.BlockSpec((_B, 256), lambda k: (0, 0)),
        out_shape=jax.ShapeDtypeStruct((_B, 256), jnp.float32),
        scratch_shapes=[pltpu.VMEM((_B, 64), jnp.float32)],
    )(triu_feats, mlp_W1, hm.reshape(_B, 2 * _HC), mlp_W1[:, _TRIU0:],
      mlp_W2, mlp_b2.reshape(1, 32), proj_W, proj_b.reshape(1, 256),
      mlp_b1.reshape(1, 64))
    return out


# speedup vs baseline: 36.0096x; 1.0024x over previous
import numpy as np
import jax
import jax.numpy as jnp
from jax.experimental import pallas as pl
from jax.experimental.pallas import tpu as pltpu

_N = 400
_B = 16
_HC = 32
_TRIU1 = _N * (_N - 1) // 2
_TRIU0 = _N * (_N + 1) // 2
_K = max(1, int(_TRIU1 * 0.05))
_EPS = 1e-5
_S = float(np.sqrt(1.0 + _EPS))
_IU0, _JU0 = np.triu_indices(_N, 0)

_KBLK = 8192
_NKB = -(-_TRIU0 // _KBLK)


def _thresh_kernel(x_ref, tn_ref):
    xb = x_ref[...]
    bits = jax.lax.bitcast_convert_type(xb, jnp.int32)
    key = bits ^ jnp.where(bits < 0, jnp.int32(0x7FFFFFFF), jnp.int32(0))
    row = jax.lax.broadcasted_iota(jnp.int32, (_B, _N, _N), 1)
    col = jax.lax.broadcasted_iota(jnp.int32, (_B, _N, _N), 2)
    neg_inf = jnp.int32(-2147483648)
    key_m = jnp.where(col > row, key, neg_inf)

    def bit_step(i, t):
        cand = t + (jnp.int32(1) << (jnp.int32(31) - i))
        cnt = jnp.sum((key_m >= cand[:, None, None]).astype(jnp.int32),
                      axis=(1, 2))
        return jnp.where(cnt >= _K, cand, t)

    t = jax.lax.fori_loop(
        0, 32, bit_step, jnp.full((_B,), neg_inf, jnp.int32))
    cnt_gt = jnp.sum((key_m > t[:, None, None]).astype(jnp.int32),
                     axis=(1, 2))
    need_eq = _K - cnt_gt
    ci = jax.lax.broadcasted_iota(jnp.int32, (_B, 8), 1)
    tn_ref[...] = jnp.where(ci == 0, t[:, None],
                            jnp.where(ci == 1, need_eq[:, None], 0))


def _graph_kernel(x_ref, tn_ref, w1_ref, b1_ref, w2_ref, b2_ref, hm_ref):
    xb = x_ref[0]
    bits = jax.lax.bitcast_convert_type(xb, jnp.int32)
    key = bits ^ jnp.where(bits < 0, jnp.int32(0x7FFFFFFF), jnp.int32(0))
    row = jax.lax.broadcasted_iota(jnp.int32, (_N, _N), 0)
    col = jax.lax.broadcasted_iota(jnp.int32, (_N, _N), 1)
    upper = col > row
    neg_inf = jnp.int32(-2147483648)
    key_m = jnp.where(upper, key, neg_inf)

    t = tn_ref[0, 0, 0]
    need_eq = tn_ref[0, 0, 1]

    eq = (key_m == t).astype(jnp.float32)
    upper_f = upper.astype(jnp.float32)
    cum_in_row = jax.lax.dot_general(
        eq, upper_f, (((1,), (0,)), ((), ())),
        preferred_element_type=jnp.float32)
    row_tot = jnp.sum(eq, axis=1)
    base = jax.lax.dot_general(
        row_tot, upper_f, (((0,), (0,)), ((), ())),
        preferred_element_type=jnp.float32)
    rank = base[:, None] + cum_in_row
    keep_tie = (eq > 0.0) & (rank < need_eq.astype(jnp.float32))
    sel = jnp.where((key_m > t) | keep_tie, 1.0, 0.0)

    asym = sel + sel.T + jnp.where(col == row, 1.0, 0.0)
    deg = jnp.sum(sel + sel.T, axis=1) + 1.0
    dinv = jax.lax.rsqrt(deg)
    anorm = dinv[:, None] * asym * dinv[None, :]

    xw1 = jax.lax.dot_general(xb, w1_ref[...], (((1,), (1,)), ((), ())),
                              preferred_element_type=jnp.float32)
    h1 = jnp.tanh(jax.lax.dot_general(anorm, xw1, (((1,), (0,)), ((), ())),
                                      preferred_element_type=jnp.float32)
                  + b1_ref[0][None, :])
    xw2 = jax.lax.dot_general(h1, w2_ref[...], (((1,), (1,)), ((), ())),
                              preferred_element_type=jnp.float32)
    h2 = jnp.tanh(jax.lax.dot_general(anorm, xw2, (((1,), (0,)), ((), ())),
                                      preferred_element_type=jnp.float32)
                  + b2_ref[0][None, :])
    hm_ref[0, 0] = jnp.concatenate(
        [jnp.mean(h1, axis=0), jnp.mean(h2, axis=0)], axis=0)


def _mlp_kernel(f_ref, w_ref, hm_ref, wh_ref, w2m_ref, b2m_ref, wp_ref,
                bp_ref, b1m_ref, out_ref, acc_ref):
    k = pl.program_id(0)

    @pl.when(k == 0)
    def _():
        acc_ref[...] = jnp.zeros_like(acc_ref)

    limit = jnp.int32(_TRIU0) - k * _KBLK
    col = jax.lax.broadcasted_iota(jnp.int32, (64, _KBLK), 1)
    wblk = jnp.where(col < limit, w_ref[...], 0.0)
    colf = jax.lax.broadcasted_iota(jnp.int32, (_B, _KBLK), 1)
    fblk = jnp.where(colf < limit, f_ref[...], 0.0)
    acc_ref[...] += jax.lax.dot_general(
        fblk, wblk, (((1,), (1,)), ((), ())),
        preferred_element_type=jnp.float32)

    @pl.when(k == _NKB - 1)
    def _():
        acc = acc_ref[...] + jax.lax.dot_general(
            hm_ref[...], wh_ref[...], (((1,), (1,)), ((), ())),
            preferred_element_type=jnp.float32)
        z1 = jax.nn.relu(acc * (1.0 / (1.0 + _EPS))
                         + b1m_ref[0][None, :] * (1.0 / _S))
        z2 = jax.lax.dot_general(z1, w2m_ref[...], (((1,), (1,)), ((), ())),
                                 preferred_element_type=jnp.float32)
        z2 = jax.nn.relu((z2 + b2m_ref[0][None, :]) * (1.0 / _S))
        z3 = jax.lax.dot_general(z2, wp_ref[...], (((1,), (1,)), ((), ())),
                                 preferred_element_type=jnp.float32)
        out_ref[...] = jax.nn.relu(z3 + bp_ref[0][None, :])


def kernel(x, W1, b1, W2, b2, mlp_W1, mlp_b1, mlp_W2, mlp_b2, proj_W, proj_b):
    tn = pl.pallas_call(
        _thresh_kernel,
        out_shape=jax.ShapeDtypeStruct((_B, 8), jnp.int32),
    )(x)

    hm = pl.pallas_call(
        _graph_kernel,
        grid=(_B,),
        in_specs=[
            pl.BlockSpec((1, _N, _N), lambda b: (b, 0, 0)),
            pl.BlockSpec((1, 1, 8), lambda b: (b, 0, 0)),
            pl.BlockSpec((_HC, _N), lambda b: (0, 0)),
            pl.BlockSpec((1, _HC), lambda b: (0, 0)),
            pl.BlockSpec((_HC, _HC), lambda b: (0, 0)),
            pl.BlockSpec((1, _HC), lambda b: (0, 0)),
        ],
        out_specs=pl.BlockSpec((1, 1, 2 * _HC), lambda b: (b, 0, 0)),
        out_shape=jax.ShapeDtypeStruct((_B, 1, 2 * _HC), jnp.float32),
    )(x, tn.reshape(_B, 1, 8), W1, b1.reshape(1, _HC), W2,
      b2.reshape(1, _HC))

    flat_idx = jnp.asarray(_IU0 * _N + _JU0, jnp.int32)
    triu_feats = jnp.take(x.reshape(_B, _N * _N), flat_idx,
                          axis=1)

    out = pl.pallas_call(
        _mlp_kernel,
        grid=(_NKB,),
        in_specs=[
            pl.BlockSpec((_B, _KBLK), lambda k: (0, k)),
            pl.BlockSpec((64, _KBLK), lambda k: (0, k)),
            pl.BlockSpec((_B, 2 * _HC), lambda k: (0, 0)),
            pl.BlockSpec((64, 2 * _HC), lambda k: (0, 0)),
            pl.BlockSpec((32, 64), lambda k: (0, 0)),
            pl.BlockSpec((1, 32), lambda k: (0, 0)),
            pl.BlockSpec((256, 32), lambda k: (0, 0)),
            pl.BlockSpec((1, 256), lambda k: (0, 0)),
            pl.BlockSpec((1, 64), lambda k: (0, 0)),
        ],
        out_specs=pl.BlockSpec((_B, 256), lambda k: (0, 0)),
        out_shape=jax.ShapeDtypeStruct((_B, 256), jnp.float32),
        scratch_shapes=[pltpu.VMEM((_B, 64), jnp.float32)],
    )(triu_feats, mlp_W1, hm.reshape(_B, 2 * _HC), mlp_W1[:, _TRIU0:],
      mlp_W2, mlp_b2.reshape(1, 32), proj_W, proj_b.reshape(1, 256),
      mlp_b1.reshape(1, 64))
    return out
